# Initial kernel scaffold; baseline (speedup 1.0000x reference)
import jax
import jax.numpy as jnp
from jax.experimental import pallas as pl


def kernel(pred_tensor, target_tensor):
    raise NotImplementedError("write your pallas kernel here")


# speedup vs baseline: 3.5836x; 3.5836x over previous
import functools

import jax
import jax.numpy as jnp
from jax import lax
from jax.experimental import pallas as pl
from jax.experimental.pallas import tpu as pltpu
from jax.experimental.pallas import tpu_sc as plsc

N_BATCH = 1024
CELLS = N_BATCH * 49
CH = 30
NC, NS, L = 2, 16, 16
NW = NC * NS
CPW = CELLS // NW
WORDS = CPW * CH
GROUPS = CPW // L

L_COORD = 5.0
L_NOOBJ = 0.5


def _newton_sqrt(x):
    i = lax.bitcast_convert_type(x, jnp.uint32)
    i = jnp.uint32(0x5F3759DF) - (i >> jnp.uint32(1))
    r = lax.bitcast_convert_type(i, jnp.float32)
    for _ in range(3):
        r = r * (1.5 - 0.5 * x * r * r)
    return x * r


def _sq(x):
    return x * x


def _cell_loss(p, t, sqrt_fn):
    conf = t[4]
    coo = jnp.where(conf > 0, 1.0, 0.0).astype(jnp.float32)
    noo = jnp.where(conf == 0, 1.0, 0.0).astype(jnp.float32)

    nooobj = noo * (_sq(p[4] - t[4]) + _sq(p[9] - t[9]))

    b2minx = t[0] - 0.5 * t[2]
    b2maxx = t[0] + 0.5 * t[2]
    b2miny = t[1] - 0.5 * t[3]
    b2maxy = t[1] + 0.5 * t[3]
    area2 = t[2] * t[3]

    def iou(off):
        bminx = p[off] - 0.5 * p[off + 2]
        bmaxx = p[off] + 0.5 * p[off + 2]
        bminy = p[off + 1] - 0.5 * p[off + 3]
        bmaxy = p[off + 1] + 0.5 * p[off + 3]
        whx = jnp.minimum(bmaxx, b2maxx) - jnp.maximum(bminx, b2minx)
        why = jnp.minimum(bmaxy, b2maxy) - jnp.maximum(bminy, b2miny)
        inter = jnp.where(whx < 0, 1.0, 0.0) * jnp.where(why < 0, 1.0, 0.0)
        area1 = p[off + 2] * p[off + 3]
        return inter / (area1 + area2 - inter)

    iou0 = iou(0)
    iou1 = iou(5)
    j1 = iou1 > iou0

    rp = [jnp.where(j1, p[5 + i], p[i]) for i in range(5)]
    rt = [jnp.where(j1, t[5 + i], t[i]) for i in range(5)]

    contain = coo * _sq(rp[4] - rt[4])
    loc_xy = _sq(rp[0] - rt[0]) + _sq(rp[1] - rt[1])
    loc_wh = (rp[2] + rt[2] - 2.0 * sqrt_fn(rp[2] * rt[2])
              + rp[3] + rt[3] - 2.0 * sqrt_fn(rp[3] * rt[3]))

    class_sq = _sq(p[10] - t[10])
    for c in range(11, CH):
        d = p[c] - t[c]
        class_sq = class_sq + d * d

    return (L_COORD * coo * (loc_xy + loc_wh) + contain
            + L_NOOBJ * nooobj + coo * class_sq)


@functools.cache
def _loss_partials_fn():
    mesh = plsc.VectorSubcoreMesh(
        core_axis_name="c", subcore_axis_name="s",
        num_cores=NC, num_subcores=NS)

    @functools.partial(
        pl.kernel,
        out_type=jax.ShapeDtypeStruct((NW, L), jnp.float32),
        mesh=mesh,
        scratch_types=[
            pltpu.VMEM((WORDS,), jnp.float32),
            pltpu.VMEM((WORDS,), jnp.float32),
            pltpu.VMEM((L,), jnp.float32),
        ],
        compiler_params=pltpu.CompilerParams(needs_layout_passes=False),
    )
    def _loss_partials(pred_hbm, tgt_hbm, out_hbm, pred_v, tgt_v, acc_v):
        wid = lax.axis_index("s") * NC + lax.axis_index("c")
        base = wid * WORDS
        pltpu.sync_copy(pred_hbm.at[pl.ds(base, WORDS)], pred_v)
        pltpu.sync_copy(tgt_hbm.at[pl.ds(base, WORDS)], tgt_v)

        iota30 = lax.iota(jnp.int32, L) * CH

        def body(g, acc):
            gb = iota30 + g * (L * CH)
            idx = [gb + c for c in range(CH)]
            p = [plsc.load_gather(pred_v, [idx[c]]) for c in range(CH)]
            t = [plsc.load_gather(tgt_v, [idx[c]]) for c in range(CH)]
            return acc + _cell_loss(p, t, _newton_sqrt)

        acc = lax.fori_loop(0, GROUPS, body, jnp.zeros((L,), jnp.float32))
        acc_v[...] = acc
        pltpu.sync_copy(acc_v, out_hbm.at[wid])

    return _loss_partials


def kernel(pred_tensor, target_tensor):
    parts = _loss_partials_fn()(
        pred_tensor.reshape(-1), target_tensor.reshape(-1))
    return jnp.sum(parts) * jnp.float32(1.0 / N_BATCH)


# speedup vs baseline: 7.1162x; 1.9858x over previous
import functools

import jax
import jax.numpy as jnp
from jax import lax
from jax.experimental import pallas as pl
from jax.experimental.pallas import tpu as pltpu
from jax.experimental.pallas import tpu_sc as plsc

N_BATCH = 1024
CELLS = 49
CH = 30
NC, NS, L = 2, 16, 16
NW = NC * NS
BCHUNK = 128
NCHUNK = N_BATCH // BCHUNK
UNITS = CELLS * NCHUNK
KMAX = -(-UNITS // NW)

L_COORD = 5.0
L_NOOBJ = 0.5


def _newton_sqrt(x):
    i = lax.bitcast_convert_type(x, jnp.uint32)
    i = jnp.uint32(0x5F3759DF) - (i >> jnp.uint32(1))
    r = lax.bitcast_convert_type(i, jnp.float32)
    for _ in range(3):
        r = r * (1.5 - 0.5 * x * r * r)
    return x * r


def _sq(x):
    return x * x


def _cell_loss(p, t, sqrt_fn):
    conf = t[4]
    coo = jnp.where(conf > 0, 1.0, 0.0).astype(jnp.float32)
    noo = jnp.where(conf == 0, 1.0, 0.0).astype(jnp.float32)

    nooobj = noo * (_sq(p[4] - t[4]) + _sq(p[9] - t[9]))

    b2minx = t[0] - 0.5 * t[2]
    b2maxx = t[0] + 0.5 * t[2]
    b2miny = t[1] - 0.5 * t[3]
    b2maxy = t[1] + 0.5 * t[3]
    area2 = t[2] * t[3]

    def iou(off):
        bminx = p[off] - 0.5 * p[off + 2]
        bmaxx = p[off] + 0.5 * p[off + 2]
        bminy = p[off + 1] - 0.5 * p[off + 3]
        bmaxy = p[off + 1] + 0.5 * p[off + 3]
        whx = jnp.minimum(bmaxx, b2maxx) - jnp.maximum(bminx, b2minx)
        why = jnp.minimum(bmaxy, b2maxy) - jnp.maximum(bminy, b2miny)
        inter = jnp.where(whx < 0, 1.0, 0.0) * jnp.where(why < 0, 1.0, 0.0)
        area1 = p[off + 2] * p[off + 3]
        return inter / (area1 + area2 - inter)

    iou0 = iou(0)
    iou1 = iou(5)
    j1 = iou1 > iou0

    rp = [jnp.where(j1, p[5 + i], p[i]) for i in range(5)]
    rt = [jnp.where(j1, t[5 + i], t[i]) for i in range(5)]

    contain = coo * _sq(rp[4] - rt[4])
    loc_xy = _sq(rp[0] - rt[0]) + _sq(rp[1] - rt[1])
    loc_wh = (rp[2] + rt[2] - 2.0 * sqrt_fn(rp[2] * rt[2])
              + rp[3] + rt[3] - 2.0 * sqrt_fn(rp[3] * rt[3]))

    class_sq = _sq(p[10] - t[10])
    for c in range(11, CH):
        d = p[c] - t[c]
        class_sq = class_sq + d * d

    return (L_COORD * coo * (loc_xy + loc_wh) + contain
            + L_NOOBJ * nooobj + coo * class_sq)


@functools.cache
def _loss_partials_fn():
    mesh = plsc.VectorSubcoreMesh(
        core_axis_name="c", subcore_axis_name="s",
        num_cores=NC, num_subcores=NS)

    @functools.partial(
        pl.kernel,
        out_type=jax.ShapeDtypeStruct((NW, L), jnp.float32),
        mesh=mesh,
        scratch_types=[
            pltpu.VMEM((CH, BCHUNK), jnp.float32),
            pltpu.VMEM((CH, BCHUNK), jnp.float32),
            pltpu.VMEM((L,), jnp.float32),
        ],
        compiler_params=pltpu.CompilerParams(needs_layout_passes=False),
    )
    def _loss_partials(pred_hbm, tgt_hbm, out_hbm, pred_v, tgt_v, acc_v):
        wid = lax.axis_index("s") * NC + lax.axis_index("c")

        def unit(k, acc):
            un = wid + NW * k
            u = jnp.minimum(un, UNITS - 1)
            cell = u >> 3
            chunk = u & (NCHUNK - 1)
            i = cell // 7
            j = cell - i * 7
            b0 = pl.multiple_of(chunk * BCHUNK, BCHUNK)
            pltpu.sync_copy(pred_hbm.at[i, j, :, pl.ds(b0, BCHUNK)], pred_v)
            pltpu.sync_copy(tgt_hbm.at[i, j, :, pl.ds(b0, BCHUNK)], tgt_v)

            def g_body(g, a):
                p = [pred_v[c, pl.ds(g * L, L)] for c in range(CH)]
                t = [tgt_v[c, pl.ds(g * L, L)] for c in range(CH)]
                return a + _cell_loss(p, t, _newton_sqrt)

            contrib = lax.fori_loop(0, BCHUNK // L, g_body,
                                    jnp.zeros((L,), jnp.float32))
            return acc + jnp.where(un < UNITS, contrib, 0.0)

        acc = lax.fori_loop(0, KMAX, unit, jnp.zeros((L,), jnp.float32))
        acc_v[...] = acc
        pltpu.sync_copy(acc_v, out_hbm.at[wid])

    return _loss_partials


def kernel(pred_tensor, target_tensor):
    pt = jnp.transpose(pred_tensor, (1, 2, 3, 0))
    tt = jnp.transpose(target_tensor, (1, 2, 3, 0))
    parts = _loss_partials_fn()(pt, tt)
    return jnp.sum(parts) * jnp.float32(1.0 / N_BATCH)


# speedup vs baseline: 8.7320x; 1.2271x over previous
import functools

import jax
import jax.numpy as jnp
from jax import lax
from jax.experimental import pallas as pl
from jax.experimental.pallas import tpu as pltpu
from jax.experimental.pallas import tpu_sc as plsc

N_BATCH = 1024
CELLS = 49
CH = 30
NC, NS, L = 2, 16, 16
NW = NC * NS
BCHUNK = 128
NCHUNK = N_BATCH // BCHUNK
UNITS = CELLS * NCHUNK
KMAX = -(-UNITS // NW)

L_COORD = 5.0
L_NOOBJ = 0.5


def _newton_sqrt(x):
    i = lax.bitcast_convert_type(x, jnp.uint32)
    i = jnp.uint32(0x5F3759DF) - (i >> jnp.uint32(1))
    r = lax.bitcast_convert_type(i, jnp.float32)
    for _ in range(2):
        r = r * (1.5 - 0.5 * x * r * r)
    return x * r


def _sq(x):
    return x * x


def _cell_loss(p, t, sqrt_fn):
    conf = t[4]
    coo = jnp.where(conf > 0, 1.0, 0.0).astype(jnp.float32)
    noo = jnp.where(conf == 0, 1.0, 0.0).astype(jnp.float32)

    nooobj = noo * (_sq(p[4] - t[4]) + _sq(p[9] - t[9]))

    b2minx = t[0] - 0.5 * t[2]
    b2maxx = t[0] + 0.5 * t[2]
    b2miny = t[1] - 0.5 * t[3]
    b2maxy = t[1] + 0.5 * t[3]
    area2 = t[2] * t[3]

    def iou(off):
        bminx = p[off] - 0.5 * p[off + 2]
        bmaxx = p[off] + 0.5 * p[off + 2]
        bminy = p[off + 1] - 0.5 * p[off + 3]
        bmaxy = p[off + 1] + 0.5 * p[off + 3]
        whx = jnp.minimum(bmaxx, b2maxx) - jnp.maximum(bminx, b2minx)
        why = jnp.minimum(bmaxy, b2maxy) - jnp.maximum(bminy, b2miny)
        inter = jnp.where(whx < 0, 1.0, 0.0) * jnp.where(why < 0, 1.0, 0.0)
        area1 = p[off + 2] * p[off + 3]
        return inter / (area1 + area2 - inter)

    iou0 = iou(0)
    iou1 = iou(5)
    j1 = iou1 > iou0

    rp = [jnp.where(j1, p[5 + i], p[i]) for i in range(5)]
    rt = [jnp.where(j1, t[5 + i], t[i]) for i in range(5)]

    contain = coo * _sq(rp[4] - rt[4])
    loc_xy = _sq(rp[0] - rt[0]) + _sq(rp[1] - rt[1])
    loc_wh = (rp[2] + rt[2] - 2.0 * sqrt_fn(rp[2] * rt[2])
              + rp[3] + rt[3] - 2.0 * sqrt_fn(rp[3] * rt[3]))

    class_sq = _sq(p[10] - t[10])
    for c in range(11, CH):
        d = p[c] - t[c]
        class_sq = class_sq + d * d

    return (L_COORD * coo * (loc_xy + loc_wh) + contain
            + L_NOOBJ * nooobj + coo * class_sq)


@functools.cache
def _loss_partials_fn():
    mesh = plsc.VectorSubcoreMesh(
        core_axis_name="c", subcore_axis_name="s",
        num_cores=NC, num_subcores=NS)

    @functools.partial(
        pl.kernel,
        out_type=jax.ShapeDtypeStruct((NW, L), jnp.float32),
        mesh=mesh,
        scratch_types=[
            pltpu.VMEM((CH, BCHUNK), jnp.float32),
            pltpu.VMEM((CH, BCHUNK), jnp.float32),
            pltpu.VMEM((CH, BCHUNK), jnp.float32),
            pltpu.VMEM((CH, BCHUNK), jnp.float32),
            pltpu.VMEM((L,), jnp.float32),
            pltpu.SemaphoreType.DMA,
            pltpu.SemaphoreType.DMA,
        ],
        compiler_params=pltpu.CompilerParams(needs_layout_passes=False),
    )
    def _loss_partials(pred_hbm, tgt_hbm, out_hbm, pred_v0, pred_v1,
                       tgt_v0, tgt_v1, acc_v, sem0, sem1):
        wid = lax.axis_index("s") * NC + lax.axis_index("c")
        sems = (sem0, sem1)
        pbufs = (pred_v0, pred_v1)
        tbufs = (tgt_v0, tgt_v1)

        def issue(k, buf):
            u = jnp.minimum(wid + NW * k, UNITS - 1)
            cell = u >> 3
            chunk = u & (NCHUNK - 1)
            i = cell // 7
            j = cell - i * 7
            b0 = pl.multiple_of(chunk * BCHUNK, BCHUNK)
            hp = pltpu.async_copy(
                pred_hbm.at[i, j, :, pl.ds(b0, BCHUNK)], pbufs[buf],
                sems[buf])
            ht = pltpu.async_copy(
                tgt_hbm.at[i, j, :, pl.ds(b0, BCHUNK)], tbufs[buf],
                sems[buf])
            return (hp, ht)

        handles = [issue(0, 0), None]
        acc = jnp.zeros((L,), jnp.float32)
        for k in range(KMAX):
            buf = k & 1
            if k + 1 < KMAX:
                handles[1 - buf] = issue(k + 1, 1 - buf)
            for h in handles[buf]:
                h.wait()

            def g_body(g, a, _buf=buf):
                p = [pbufs[_buf][c, pl.ds(g * L, L)] for c in range(CH)]
                t = [tbufs[_buf][c, pl.ds(g * L, L)] for c in range(CH)]
                return a + _cell_loss(p, t, _newton_sqrt)

            contrib = lax.fori_loop(0, BCHUNK // L, g_body,
                                    jnp.zeros((L,), jnp.float32))
            acc = acc + jnp.where(wid + NW * k < UNITS, contrib, 0.0)

        acc_v[...] = acc
        pltpu.sync_copy(acc_v, out_hbm.at[wid])

    return _loss_partials


def kernel(pred_tensor, target_tensor):
    pt = jnp.transpose(pred_tensor, (1, 2, 3, 0))
    tt = jnp.transpose(target_tensor, (1, 2, 3, 0))
    parts = _loss_partials_fn()(pt, tt)
    return jnp.sum(parts) * jnp.float32(1.0 / N_BATCH)
